# baseline (speedup 1.0000x reference)
import functools

import jax
import jax.numpy as jnp
from jax import lax
from jax.experimental import pallas as pl
from jax.experimental.pallas import tpu as pltpu
from jax.experimental.pallas import tpu_sc as plsc

B, L, E = 128, 512, 768
VOCAB = 30522
E2 = E // 2
NC, NS = 2, 16
NW = NC * NS
TYPE_ROWS = 2
LN_BATCHES = 4
CAST_ROWS = 1024

CHUNKS = (8, 40, 40, 40)
CHUNK_R = (64, 80, 80, 80)


def _cast_table(word_emb):
    def body(x_ref, o_ref):
        x = x_ref[...]
        u = lax.bitcast_convert_type(x, jnp.int32)
        r = u + jnp.int32(0x7FFF) + lax.bitwise_and(
            lax.shift_right_logical(u, 16), jnp.int32(1)
        )
        r = lax.shift_right_logical(r, 16)
        lo = r[:, :E2]
        hi = r[:, E2:]
        o_ref[...] = lax.bitwise_or(lo, lax.shift_left(hi, 16))

    return pl.pallas_call(
        body,
        grid=(pl.cdiv(VOCAB, CAST_ROWS),),
        in_specs=[pl.BlockSpec((CAST_ROWS, E), lambda i: (i, 0))],
        out_specs=pl.BlockSpec((CAST_ROWS, E2), lambda i: (i, 0)),
        out_shape=jax.ShapeDtypeStruct((VOCAB, E2), jnp.int32),
    )(word_emb)


def _sc_gather(ids_flat, packed_table, tok_base, tok_c, r):
    mesh = plsc.VectorSubcoreMesh(core_axis_name="c", subcore_axis_name="s")
    tpw = tok_c // NW
    nch = tpw // r
    assert nch >= 2 and nch % 2 == 0 and tpw % r == 0 and r <= 128

    @functools.partial(
        pl.kernel,
        mesh=mesh,
        out_type=jax.ShapeDtypeStruct((tok_c, E2), jnp.int32),
        scratch_types=[
            pltpu.VMEM((tpw,), jnp.int32),
            pltpu.VMEM((2, r, E2), jnp.int32),
            pltpu.SemaphoreType.DMA,
            pltpu.SemaphoreType.DMA,
            pltpu.SemaphoreType.DMA,
            pltpu.SemaphoreType.DMA,
        ],
    )
    def k(ids_hbm, word_hbm, out_hbm, idx_v, rows_v, g0, g1, s0, s1):
        wid = lax.axis_index("s") * NC + lax.axis_index("c")
        base = pl.multiple_of(wid * tpw, tpw)
        pltpu.sync_copy(ids_hbm.at[pl.ds(tok_base + base, tpw)], idx_v)
        gs = (g0, g1)
        ss = (s0, s1)

        def gather(c, slot):
            off = pl.multiple_of(c * r, r)
            return pltpu.make_async_copy(
                word_hbm.at[idx_v.at[pl.ds(off, r)]], rows_v.at[slot], gs[slot]
            )

        def store(c, slot):
            off = pl.multiple_of(base + c * r, r)
            return pltpu.make_async_copy(
                rows_v.at[slot], out_hbm.at[pl.ds(off, r)], ss[slot]
            )

        gather(0, 0).start()
        gather(1, 1).start()

        def step(i, carry):
            c0 = 2 * i
            gather(c0, 0).wait()
            store(c0, 0).start()
            gather(c0 + 1, 1).wait()
            store(c0 + 1, 1).start()
            store(c0, 0).wait()

            @pl.when(c0 + 2 < nch)
            def _():
                gather(c0 + 2, 0).start()

            store(c0 + 1, 1).wait()

            @pl.when(c0 + 3 < nch)
            def _():
                gather(c0 + 3, 1).start()

            return carry

        lax.fori_loop(0, nch // 2, step, 0)

    return k(ids_flat, packed_table)


def _tc_layernorm(gathered_c, pos_emb, seg_emb, gamma2d, beta2d, prev, batch0, bc):
    def body(*refs):
        if prev is None:
            x_ref, pos_ref, seg_ref, g_ref, b_ref, o_ref = refs
        else:
            x_ref, pos_ref, seg_ref, g_ref, b_ref, _prev_ref, o_ref = refs
        p = x_ref[...]
        lo = lax.bitcast_convert_type(lax.shift_left(p, 16), jnp.float32)
        hi = lax.bitcast_convert_type(
            lax.bitwise_and(p, jnp.int32(-65536)), jnp.float32
        )
        w = jnp.concatenate([lo, hi], axis=-1).reshape(LN_BATCHES, L, E)
        x = w + pos_ref[...][None] + seg_ref[0, :][None, None, :]
        mean = jnp.mean(x, axis=-1, keepdims=True)
        xc = x - mean
        var = jnp.mean(xc * xc, axis=-1, keepdims=True)
        y = xc * lax.rsqrt(var + 1e-12) * g_ref[0, :][None, None, :] + b_ref[0, :][None, None, :]
        o_ref[...] = y.reshape(LN_BATCHES * L, E)

    ln_steps = bc // LN_BATCHES
    step0 = batch0 // LN_BATCHES
    in_specs = [
        pl.BlockSpec((LN_BATCHES * L, E2), lambda i: (i, 0)),
        pl.BlockSpec((L, E), lambda i: (0, 0)),
        pl.BlockSpec((TYPE_ROWS, E), lambda i: (0, 0)),
        pl.BlockSpec((1, E), lambda i: (0, 0)),
        pl.BlockSpec((1, E), lambda i: (0, 0)),
    ]
    args = [gathered_c, pos_emb, seg_emb, gamma2d, beta2d]
    kwargs = {}
    if prev is not None:
        in_specs.append(pl.BlockSpec((8, E), lambda i: (0, 0)))
        args.append(prev)
        kwargs["input_output_aliases"] = {5: 0}
    return pl.pallas_call(
        body,
        grid=(ln_steps,),
        in_specs=in_specs,
        out_specs=pl.BlockSpec((LN_BATCHES * L, E), lambda i: (step0 + i, 0)),
        out_shape=jax.ShapeDtypeStruct((B * L, E), jnp.float32),
        **kwargs,
    )(*args)


def kernel(inputs, word_emb, seg_emb, pos_emb, gamma, beta):
    ids_flat = inputs.reshape(-1).astype(jnp.int32)
    gamma2d = gamma.reshape(1, E)
    beta2d = beta.reshape(1, E)
    packed = _cast_table(word_emb)
    out = None
    batch0 = 0
    for bc, r in zip(CHUNKS, CHUNK_R):
        gathered_c = _sc_gather(ids_flat, packed, batch0 * L, bc * L, r)
        out = _tc_layernorm(
            gathered_c, pos_emb, seg_emb, gamma2d, beta2d, out, batch0, bc
        )
        batch0 += bc
    return out.reshape(B, L, E)


# speedup vs baseline: 1.0126x; 1.0126x over previous
import functools

import jax
import jax.numpy as jnp
from jax import lax
from jax.experimental import pallas as pl
from jax.experimental.pallas import tpu as pltpu
from jax.experimental.pallas import tpu_sc as plsc

B, L, E = 128, 512, 768
VOCAB = 30522
E2 = E // 2
NC, NS = 2, 16
NW = NC * NS
TYPE_ROWS = 2
LN_BATCHES = 4
CAST_ROWS = 1024

CHUNKS = (32, 32, 32, 32)
CHUNK_R = (128, 128, 128, 128)


def _cast_table(word_emb):
    def body(x_ref, o_ref):
        x = x_ref[...]
        u = lax.bitcast_convert_type(x, jnp.int32)
        r = u + jnp.int32(0x7FFF) + lax.bitwise_and(
            lax.shift_right_logical(u, 16), jnp.int32(1)
        )
        r = lax.shift_right_logical(r, 16)
        lo = r[:, :E2]
        hi = r[:, E2:]
        o_ref[...] = lax.bitwise_or(lo, lax.shift_left(hi, 16))

    return pl.pallas_call(
        body,
        grid=(pl.cdiv(VOCAB, CAST_ROWS),),
        in_specs=[pl.BlockSpec((CAST_ROWS, E), lambda i: (i, 0))],
        out_specs=pl.BlockSpec((CAST_ROWS, E2), lambda i: (i, 0)),
        out_shape=jax.ShapeDtypeStruct((VOCAB, E2), jnp.int32),
    )(word_emb)


def _sc_gather(ids_flat, packed_table, tok_base, tok_c, r):
    mesh = plsc.VectorSubcoreMesh(core_axis_name="c", subcore_axis_name="s")
    tpw = tok_c // NW
    nch = tpw // r
    assert nch >= 2 and nch % 2 == 0 and tpw % r == 0 and r <= 128

    @functools.partial(
        pl.kernel,
        mesh=mesh,
        out_type=jax.ShapeDtypeStruct((tok_c, E2), jnp.int32),
        scratch_types=[
            pltpu.VMEM((tpw,), jnp.int32),
            pltpu.VMEM((2, r, E2), jnp.int32),
            pltpu.SemaphoreType.DMA,
            pltpu.SemaphoreType.DMA,
            pltpu.SemaphoreType.DMA,
            pltpu.SemaphoreType.DMA,
        ],
    )
    def k(ids_hbm, word_hbm, out_hbm, idx_v, rows_v, g0, g1, s0, s1):
        wid = lax.axis_index("s") * NC + lax.axis_index("c")
        base = pl.multiple_of(wid * tpw, tpw)
        pltpu.sync_copy(ids_hbm.at[pl.ds(tok_base + base, tpw)], idx_v)
        gs = (g0, g1)
        ss = (s0, s1)

        def gather(c, slot):
            off = pl.multiple_of(c * r, r)
            return pltpu.make_async_copy(
                word_hbm.at[idx_v.at[pl.ds(off, r)]], rows_v.at[slot], gs[slot]
            )

        def store(c, slot):
            off = pl.multiple_of(base + c * r, r)
            return pltpu.make_async_copy(
                rows_v.at[slot], out_hbm.at[pl.ds(off, r)], ss[slot]
            )

        gather(0, 0).start()
        gather(1, 1).start()

        def step(i, carry):
            c0 = 2 * i
            gather(c0, 0).wait()
            store(c0, 0).start()
            gather(c0 + 1, 1).wait()
            store(c0 + 1, 1).start()
            store(c0, 0).wait()

            @pl.when(c0 + 2 < nch)
            def _():
                gather(c0 + 2, 0).start()

            store(c0 + 1, 1).wait()

            @pl.when(c0 + 3 < nch)
            def _():
                gather(c0 + 3, 1).start()

            return carry

        lax.fori_loop(0, nch // 2, step, 0)

    return k(ids_flat, packed_table)


def _tc_layernorm(gathered_c, pos_emb, seg_emb, gamma2d, beta2d, prev, batch0, bc):
    def body(*refs):
        if prev is None:
            x_ref, pos_ref, seg_ref, g_ref, b_ref, o_ref = refs
        else:
            x_ref, pos_ref, seg_ref, g_ref, b_ref, _prev_ref, o_ref = refs
        p = x_ref[...]
        lo = lax.bitcast_convert_type(lax.shift_left(p, 16), jnp.float32)
        hi = lax.bitcast_convert_type(
            lax.bitwise_and(p, jnp.int32(-65536)), jnp.float32
        )
        w = jnp.concatenate([lo, hi], axis=-1).reshape(LN_BATCHES, L, E)
        x = w + pos_ref[...][None] + seg_ref[0, :][None, None, :]
        mean = jnp.mean(x, axis=-1, keepdims=True)
        xc = x - mean
        var = jnp.mean(xc * xc, axis=-1, keepdims=True)
        y = xc * lax.rsqrt(var + 1e-12) * g_ref[0, :][None, None, :] + b_ref[0, :][None, None, :]
        o_ref[...] = y.reshape(LN_BATCHES * L, E)

    ln_steps = bc // LN_BATCHES
    step0 = batch0 // LN_BATCHES
    in_specs = [
        pl.BlockSpec((LN_BATCHES * L, E2), lambda i: (i, 0)),
        pl.BlockSpec((L, E), lambda i: (0, 0)),
        pl.BlockSpec((TYPE_ROWS, E), lambda i: (0, 0)),
        pl.BlockSpec((1, E), lambda i: (0, 0)),
        pl.BlockSpec((1, E), lambda i: (0, 0)),
    ]
    args = [gathered_c, pos_emb, seg_emb, gamma2d, beta2d]
    kwargs = {}
    if prev is not None:
        in_specs.append(pl.BlockSpec((8, E), lambda i: (0, 0)))
        args.append(prev)
        kwargs["input_output_aliases"] = {5: 0}
    return pl.pallas_call(
        body,
        grid=(ln_steps,),
        in_specs=in_specs,
        out_specs=pl.BlockSpec((LN_BATCHES * L, E), lambda i: (step0 + i, 0)),
        out_shape=jax.ShapeDtypeStruct((B * L, E), jnp.float32),
        **kwargs,
    )(*args)


def kernel(inputs, word_emb, seg_emb, pos_emb, gamma, beta):
    ids_flat = inputs.reshape(-1).astype(jnp.int32)
    gamma2d = gamma.reshape(1, E)
    beta2d = beta.reshape(1, E)
    packed = _cast_table(word_emb)
    out = None
    batch0 = 0
    for bc, r in zip(CHUNKS, CHUNK_R):
        gathered_c = _sc_gather(ids_flat, packed, batch0 * L, bc * L, r)
        out = _tc_layernorm(
            gathered_c, pos_emb, seg_emb, gamma2d, beta2d, out, batch0, bc
        )
        batch0 += bc
    return out.reshape(B, L, E)


# speedup vs baseline: 1.0330x; 1.0201x over previous
import functools

import jax
import jax.numpy as jnp
from jax import lax
from jax.experimental import pallas as pl
from jax.experimental.pallas import tpu as pltpu
from jax.experimental.pallas import tpu_sc as plsc

B, L, E = 128, 512, 768
VOCAB = 30522
E2 = E // 2
NC, NS = 2, 16
NW = NC * NS
TYPE_ROWS = 2
LN_BATCHES = 8
CAST_ROWS = 1024

CHUNKS = (32, 32, 32, 32)
CHUNK_R = (128, 128, 128, 128)


def _cast_table(word_emb):
    def body(x_ref, o_ref):
        x = x_ref[...]
        u = lax.bitcast_convert_type(x, jnp.int32)
        r = u + jnp.int32(0x7FFF) + lax.bitwise_and(
            lax.shift_right_logical(u, 16), jnp.int32(1)
        )
        r = lax.shift_right_logical(r, 16)
        lo = r[:, :E2]
        hi = r[:, E2:]
        o_ref[...] = lax.bitwise_or(lo, lax.shift_left(hi, 16))

    return pl.pallas_call(
        body,
        grid=(pl.cdiv(VOCAB, CAST_ROWS),),
        in_specs=[pl.BlockSpec((CAST_ROWS, E), lambda i: (i, 0))],
        out_specs=pl.BlockSpec((CAST_ROWS, E2), lambda i: (i, 0)),
        out_shape=jax.ShapeDtypeStruct((VOCAB, E2), jnp.int32),
    )(word_emb)


def _sc_gather(ids_flat, packed_table, tok_base, tok_c, r):
    mesh = plsc.VectorSubcoreMesh(core_axis_name="c", subcore_axis_name="s")
    tpw = tok_c // NW
    nch = tpw // r
    assert nch >= 2 and nch % 2 == 0 and tpw % r == 0 and r <= 128

    @functools.partial(
        pl.kernel,
        mesh=mesh,
        out_type=jax.ShapeDtypeStruct((tok_c, E2), jnp.int32),
        scratch_types=[
            pltpu.VMEM((tpw,), jnp.int32),
            pltpu.VMEM((2, r, E2), jnp.int32),
            pltpu.SemaphoreType.DMA,
            pltpu.SemaphoreType.DMA,
            pltpu.SemaphoreType.DMA,
            pltpu.SemaphoreType.DMA,
        ],
    )
    def k(ids_hbm, word_hbm, out_hbm, idx_v, rows_v, g0, g1, s0, s1):
        wid = lax.axis_index("s") * NC + lax.axis_index("c")
        base = pl.multiple_of(wid * tpw, tpw)
        pltpu.sync_copy(ids_hbm.at[pl.ds(tok_base + base, tpw)], idx_v)
        gs = (g0, g1)
        ss = (s0, s1)

        def gather(c, slot):
            off = pl.multiple_of(c * r, r)
            return pltpu.make_async_copy(
                word_hbm.at[idx_v.at[pl.ds(off, r)]], rows_v.at[slot], gs[slot]
            )

        def store(c, slot):
            off = pl.multiple_of(base + c * r, r)
            return pltpu.make_async_copy(
                rows_v.at[slot], out_hbm.at[pl.ds(off, r)], ss[slot]
            )

        gather(0, 0).start()
        gather(1, 1).start()

        def step(i, carry):
            c0 = 2 * i
            gather(c0, 0).wait()
            store(c0, 0).start()
            gather(c0 + 1, 1).wait()
            store(c0 + 1, 1).start()
            store(c0, 0).wait()

            @pl.when(c0 + 2 < nch)
            def _():
                gather(c0 + 2, 0).start()

            store(c0 + 1, 1).wait()

            @pl.when(c0 + 3 < nch)
            def _():
                gather(c0 + 3, 1).start()

            return carry

        lax.fori_loop(0, nch // 2, step, 0)

    return k(ids_flat, packed_table)


def _tc_layernorm(gathered_c, pos_emb, seg_emb, gamma2d, beta2d, prev, batch0, bc):
    def body(*refs):
        if prev is None:
            x_ref, pos_ref, seg_ref, g_ref, b_ref, o_ref = refs
        else:
            x_ref, pos_ref, seg_ref, g_ref, b_ref, _prev_ref, o_ref = refs
        p = x_ref[...]
        lo = lax.bitcast_convert_type(lax.shift_left(p, 16), jnp.float32)
        hi = lax.bitcast_convert_type(
            lax.bitwise_and(p, jnp.int32(-65536)), jnp.float32
        )
        w = jnp.concatenate([lo, hi], axis=-1).reshape(LN_BATCHES, L, E)
        x = w + pos_ref[...][None] + seg_ref[0, :][None, None, :]
        mean = jnp.mean(x, axis=-1, keepdims=True)
        xc = x - mean
        var = jnp.mean(xc * xc, axis=-1, keepdims=True)
        y = xc * lax.rsqrt(var + 1e-12) * g_ref[0, :][None, None, :] + b_ref[0, :][None, None, :]
        o_ref[...] = y.reshape(LN_BATCHES * L, E)

    ln_steps = bc // LN_BATCHES
    step0 = batch0 // LN_BATCHES
    in_specs = [
        pl.BlockSpec((LN_BATCHES * L, E2), lambda i: (i, 0)),
        pl.BlockSpec((L, E), lambda i: (0, 0)),
        pl.BlockSpec((TYPE_ROWS, E), lambda i: (0, 0)),
        pl.BlockSpec((1, E), lambda i: (0, 0)),
        pl.BlockSpec((1, E), lambda i: (0, 0)),
    ]
    args = [gathered_c, pos_emb, seg_emb, gamma2d, beta2d]
    kwargs = {}
    if prev is not None:
        in_specs.append(pl.BlockSpec((8, E), lambda i: (0, 0)))
        args.append(prev)
        kwargs["input_output_aliases"] = {5: 0}
    return pl.pallas_call(
        body,
        grid=(ln_steps,),
        in_specs=in_specs,
        out_specs=pl.BlockSpec((LN_BATCHES * L, E), lambda i: (step0 + i, 0)),
        out_shape=jax.ShapeDtypeStruct((B * L, E), jnp.float32),
        **kwargs,
    )(*args)


def kernel(inputs, word_emb, seg_emb, pos_emb, gamma, beta):
    ids_flat = inputs.reshape(-1).astype(jnp.int32)
    gamma2d = gamma.reshape(1, E)
    beta2d = beta.reshape(1, E)
    packed = _cast_table(word_emb)
    out = None
    batch0 = 0
    for bc, r in zip(CHUNKS, CHUNK_R):
        gathered_c = _sc_gather(ids_flat, packed, batch0 * L, bc * L, r)
        out = _tc_layernorm(
            gathered_c, pos_emb, seg_emb, gamma2d, beta2d, out, batch0, bc
        )
        batch0 += bc
    return out.reshape(B, L, E)
